# baseline (device time: 147776 ns/iter reference)
import jax
import jax.numpy as jnp
from jax import lax
from jax.experimental import pallas as pl
from jax.experimental.pallas import tpu as pltpu

N_DEV = 16
NH = N_DEV - 1


def kernel(x, w_mat):
    m, k = x.shape
    _, n = w_mat.shape
    ch = m // N_DEV

    def body(x_ref, w_ref, out_ref, rs_buf, rs_send, rs_recv, ag_send, ag_recv):
        my = lax.axis_index("i")
        left = lax.rem(my + N_DEV - 1, N_DEV)
        right = lax.rem(my + 1, N_DEV)

        barrier = pltpu.get_barrier_semaphore()
        for nbr in (left, right):
            pl.semaphore_signal(
                barrier, inc=1, device_id=(nbr,),
                device_id_type=pl.DeviceIdType.MESH,
            )
        pl.semaphore_wait(barrier, 2)

        out_ref[:, :] = jnp.dot(
            x_ref[:, :], w_ref[:, :], preferred_element_type=jnp.float32
        )

        for h in range(NH):
            s_idx = lax.rem(my - h + N_DEV, N_DEV)
            rdma = pltpu.make_async_remote_copy(
                src_ref=out_ref.at[pl.ds(s_idx * ch, ch), :],
                dst_ref=rs_buf.at[h],
                send_sem=rs_send.at[h],
                recv_sem=rs_recv.at[h],
                device_id=(right,),
                device_id_type=pl.DeviceIdType.MESH,
            )
            rdma.start()
            rdma.wait()
            r_idx = lax.rem(my - h - 1 + N_DEV, N_DEV)
            row = pl.ds(r_idx * ch, ch)
            out_ref[row, :] = out_ref[row, :] + rs_buf[h]

        for h in range(NH):
            g_idx = lax.rem(my + 1 - h + N_DEV, N_DEV)
            row = pl.ds(g_idx * ch, ch)
            rdma = pltpu.make_async_remote_copy(
                src_ref=out_ref.at[row, :],
                dst_ref=out_ref.at[row, :],
                send_sem=ag_send.at[h],
                recv_sem=ag_recv.at[h],
                device_id=(right,),
                device_id_type=pl.DeviceIdType.MESH,
            )
            rdma.start()
            rdma.wait()

    return pl.pallas_call(
        body,
        out_shape=jax.ShapeDtypeStruct((m, n), jnp.float32),
        in_specs=[
            pl.BlockSpec(memory_space=pltpu.VMEM),
            pl.BlockSpec(memory_space=pltpu.VMEM),
        ],
        out_specs=pl.BlockSpec(memory_space=pltpu.VMEM),
        scratch_shapes=[
            pltpu.VMEM((NH, ch, n), jnp.float32),
            pltpu.SemaphoreType.DMA((NH,)),
            pltpu.SemaphoreType.DMA((NH,)),
            pltpu.SemaphoreType.DMA((NH,)),
            pltpu.SemaphoreType.DMA((NH,)),
        ],
        compiler_params=pltpu.CompilerParams(collective_id=0),
    )(x, w_mat)


# device time: 71512 ns/iter; 2.0665x vs baseline; 2.0665x over previous
import jax
import jax.numpy as jnp
from jax import lax
from jax.experimental import pallas as pl
from jax.experimental.pallas import tpu as pltpu

N_DEV = 16

_RSX, _RSY = 0, 1
_RSZ = 2
_AGZ = 5
_AGY, _AGX = 8, 9
_NSEM = 10


def kernel(x, w_mat):
    m, _ = x.shape
    _, n = w_mat.shape
    h2 = m // 2
    h4 = m // 4
    ch = m // N_DEV

    def body(x_ref, w_ref, out_ref, sbuf, rbx, rby, rbz, rbza, rbya, rbxa,
             send_sems, recv_sems, exit_sem):
        p = lax.axis_index("i")
        q = lax.rem(p, 4)
        z = lax.div(p, 4)
        xbit = lax.rem(q, 2) ^ lax.div(q, 2)
        ybit = lax.div(q, 2)
        px = p + 1 - 2 * (lax.rem(q, 2))
        py = p + 3 - 2 * q
        zright = lax.rem(p + 4, N_DEV)
        zleft = lax.rem(p + 12, N_DEV)

        partners = (px, py, zleft, zright)
        barrier = pltpu.get_barrier_semaphore()
        for nbr in partners:
            pl.semaphore_signal(
                barrier, inc=1, device_id=(nbr,),
                device_id_type=pl.DeviceIdType.MESH,
            )
        pl.semaphore_wait(barrier, len(partners))

        out_ref[:, :] = jnp.dot(
            x_ref[:, :], w_ref[:, :], preferred_element_type=jnp.float32
        )

        ox = xbit * h2
        oxy = ox + ybit * h4

        def exchange(slot, src, dst, peer):
            rdma = pltpu.make_async_remote_copy(
                src_ref=src, dst_ref=dst,
                send_sem=send_sems.at[slot], recv_sem=recv_sems.at[slot],
                device_id=(peer,), device_id_type=pl.DeviceIdType.MESH,
            )
            rdma.start()
            rdma.wait()

        sx = (1 - xbit) * h2
        sbuf[pl.ds(0, h2), :] = out_ref[pl.ds(sx, h2), :].astype(jnp.bfloat16)
        exchange(_RSX, sbuf.at[pl.ds(0, h2), :], rbx, px)
        keep = pl.ds(ox, h2)
        out_ref[keep, :] = out_ref[keep, :] + rbx[:, :].astype(jnp.float32)

        sy = ox + (1 - ybit) * h4
        sbuf[pl.ds(0, h4), :] = out_ref[pl.ds(sy, h4), :].astype(jnp.bfloat16)
        exchange(_RSY, sbuf.at[pl.ds(0, h4), :], rby, py)
        keep = pl.ds(oxy, h4)
        out_ref[keep, :] = out_ref[keep, :] + rby[:, :].astype(jnp.float32)

        for h in range(3):
            s = lax.rem(z - h + 4, 4)
            sbuf[pl.ds(0, ch), :] = out_ref[
                pl.ds(oxy + s * ch, ch), :
            ].astype(jnp.bfloat16)
            exchange(_RSZ + h, sbuf.at[pl.ds(0, ch), :], rbz.at[h], zright)
            r = lax.rem(z - h + 3, 4)
            keep = pl.ds(oxy + r * ch, ch)
            out_ref[keep, :] = out_ref[keep, :] + rbz[h].astype(jnp.float32)

        g0 = lax.rem(z + 1, 4)
        sbuf[pl.ds(0, ch), :] = out_ref[
            pl.ds(oxy + g0 * ch, ch), :
        ].astype(jnp.bfloat16)
        exchange(_AGZ, sbuf.at[pl.ds(0, ch), :], rbza.at[0], zright)
        r = z
        out_ref[pl.ds(oxy + r * ch, ch), :] = rbza[0].astype(jnp.float32)
        for h in range(1, 3):
            exchange(_AGZ + h, rbza.at[h - 1], rbza.at[h], zright)
            r = lax.rem(z - h + 4, 4)
            out_ref[pl.ds(oxy + r * ch, ch), :] = rbza[h].astype(jnp.float32)

        sbuf[pl.ds(0, h4), :] = out_ref[pl.ds(oxy, h4), :].astype(jnp.bfloat16)
        exchange(_AGY, sbuf.at[pl.ds(0, h4), :], rbya, py)
        out_ref[pl.ds(sy, h4), :] = rbya[:, :].astype(jnp.float32)

        sbuf[pl.ds(0, h2), :] = out_ref[pl.ds(ox, h2), :].astype(jnp.bfloat16)
        exchange(_AGX, sbuf.at[pl.ds(0, h2), :], rbxa, px)
        out_ref[pl.ds(sx, h2), :] = rbxa[:, :].astype(jnp.float32)

        for nbr in partners:
            pl.semaphore_signal(
                exit_sem, inc=1, device_id=(nbr,),
                device_id_type=pl.DeviceIdType.MESH,
            )
        pl.semaphore_wait(exit_sem, len(partners))

    return pl.pallas_call(
        body,
        out_shape=jax.ShapeDtypeStruct((m, n), jnp.float32),
        in_specs=[
            pl.BlockSpec(memory_space=pltpu.VMEM),
            pl.BlockSpec(memory_space=pltpu.VMEM),
        ],
        out_specs=pl.BlockSpec(memory_space=pltpu.VMEM),
        scratch_shapes=[
            pltpu.VMEM((h2, n), jnp.bfloat16),
            pltpu.VMEM((h2, n), jnp.bfloat16),
            pltpu.VMEM((h4, n), jnp.bfloat16),
            pltpu.VMEM((3, ch, n), jnp.bfloat16),
            pltpu.VMEM((3, ch, n), jnp.bfloat16),
            pltpu.VMEM((h4, n), jnp.bfloat16),
            pltpu.VMEM((h2, n), jnp.bfloat16),
            pltpu.SemaphoreType.DMA((_NSEM,)),
            pltpu.SemaphoreType.DMA((_NSEM,)),
            pltpu.SemaphoreType.REGULAR,
        ],
        compiler_params=pltpu.CompilerParams(collective_id=0),
    )(x, w_mat)


# device time: 58621 ns/iter; 2.5209x vs baseline; 1.2199x over previous
import jax
import jax.numpy as jnp
from jax import lax
from jax.experimental import pallas as pl
from jax.experimental.pallas import tpu as pltpu

N_DEV = 16
NCH = 4
NT = 10


def kernel(x, w_mat):
    m, _ = x.shape
    _, n = w_mat.shape
    h2 = m // 2
    h4 = m // 4
    ch = m // N_DEV
    cw = n // NCH

    def body(x_ref, w_ref, out_ref, sbuf, rbx, rby, rbz, rbza, rbya, rbxa,
             send_sems, recv_sems, exit_sem):
        p = lax.axis_index("i")
        q = lax.rem(p, 4)
        z = lax.div(p, 4)
        xbit = lax.rem(q, 2) ^ lax.div(q, 2)
        ybit = lax.div(q, 2)
        px = p + 1 - 2 * (lax.rem(q, 2))
        py = p + 3 - 2 * q
        zright = lax.rem(p + 4, N_DEV)

        ox = xbit * h2
        oxy = ox + ybit * h4
        sx = (1 - xbit) * h2
        sy = ox + (1 - ybit) * h4

        partners = (px, py, lax.rem(p + 12, N_DEV), zright)
        barrier = pltpu.get_barrier_semaphore()
        for nbr in partners:
            pl.semaphore_signal(
                barrier, inc=1, device_id=(nbr,),
                device_id_type=pl.DeviceIdType.MESH,
            )
        pl.semaphore_wait(barrier, len(partners))

        out_ref[:, :] = jnp.dot(
            x_ref[:, :], w_ref[:, :], preferred_element_type=jnp.float32
        )

        f32 = jnp.float32
        bf16 = jnp.bfloat16
        rdmas = {}

        def start(c, k, src, dst, peer):
            d = pltpu.make_async_remote_copy(
                src_ref=src, dst_ref=dst,
                send_sem=send_sems.at[c, k], recv_sem=recv_sems.at[c, k],
                device_id=(peer,), device_id_type=pl.DeviceIdType.MESH,
            )
            d.start()
            rdmas[(c, k)] = d

        def stage_start(c, k, rows, nrows, dst, peer):
            co = pl.ds(c * cw, cw)
            sbuf[c, pl.ds(0, nrows), :] = out_ref[
                pl.ds(rows, nrows), co
            ].astype(bf16)
            start(c, k, sbuf.at[c, pl.ds(0, nrows), :], dst, peer)

        def issue(c, k):
            if k == 0:
                stage_start(c, k, sx, h2, rbx.at[c], px)
            elif k == 1:
                stage_start(c, k, sy, h4, rby.at[c], py)
            elif k in (2, 3, 4):
                h = k - 2
                s = lax.rem(z - h + 4, 4)
                stage_start(c, k, oxy + s * ch, ch, rbz.at[c, h], zright)
            elif k == 5:
                g0 = lax.rem(z + 1, 4)
                stage_start(c, k, oxy + g0 * ch, ch, rbza.at[c, 0], zright)
            elif k in (6, 7):
                h = k - 5
                start(c, k, rbza.at[c, h - 1], rbza.at[c, h], zright)
            elif k == 8:
                stage_start(c, k, oxy, h4, rbya.at[c], py)
            else:
                stage_start(c, k, ox, h2, rbxa.at[c], px)

        def apply(c, k):
            rdmas.pop((c, k)).wait()
            co = pl.ds(c * cw, cw)
            if k == 0:
                rows = pl.ds(ox, h2)
                out_ref[rows, co] = out_ref[rows, co] + rbx[c].astype(f32)
            elif k == 1:
                rows = pl.ds(oxy, h4)
                out_ref[rows, co] = out_ref[rows, co] + rby[c].astype(f32)
            elif k in (2, 3, 4):
                h = k - 2
                r = lax.rem(z - h + 3, 4)
                rows = pl.ds(oxy + r * ch, ch)
                out_ref[rows, co] = out_ref[rows, co] + rbz[c, h].astype(f32)
            elif k in (5, 6, 7):
                h = k - 5
                r = lax.rem(z - h + 4, 4)
                out_ref[pl.ds(oxy + r * ch, ch), co] = rbza[c, h].astype(f32)
            elif k == 8:
                out_ref[pl.ds(sy, h4), co] = rbya[c].astype(f32)
            else:
                out_ref[pl.ds(sx, h2), co] = rbxa[c].astype(f32)

        for t in range(NCH + NT):
            for c in range(NCH):
                k = t - c
                if 0 < k <= NT:
                    apply(c, k - 1)
                if 0 <= k < NT:
                    issue(c, k)

        for nbr in partners:
            pl.semaphore_signal(
                exit_sem, inc=1, device_id=(nbr,),
                device_id_type=pl.DeviceIdType.MESH,
            )
        pl.semaphore_wait(exit_sem, len(partners))

    return pl.pallas_call(
        body,
        out_shape=jax.ShapeDtypeStruct((m, n), jnp.float32),
        in_specs=[
            pl.BlockSpec(memory_space=pltpu.VMEM),
            pl.BlockSpec(memory_space=pltpu.VMEM),
        ],
        out_specs=pl.BlockSpec(memory_space=pltpu.VMEM),
        scratch_shapes=[
            pltpu.VMEM((NCH, h2, cw), jnp.bfloat16),
            pltpu.VMEM((NCH, h2, cw), jnp.bfloat16),
            pltpu.VMEM((NCH, h4, cw), jnp.bfloat16),
            pltpu.VMEM((NCH, 3, ch, cw), jnp.bfloat16),
            pltpu.VMEM((NCH, 3, ch, cw), jnp.bfloat16),
            pltpu.VMEM((NCH, h4, cw), jnp.bfloat16),
            pltpu.VMEM((NCH, h2, cw), jnp.bfloat16),
            pltpu.SemaphoreType.DMA((NCH, NT)),
            pltpu.SemaphoreType.DMA((NCH, NT)),
            pltpu.SemaphoreType.REGULAR,
        ],
        compiler_params=pltpu.CompilerParams(collective_id=0),
    )(x, w_mat)


# device time: 52446 ns/iter; 2.8177x vs baseline; 1.1177x over previous
import jax
import jax.numpy as jnp
from jax import lax
from jax.experimental import pallas as pl
from jax.experimental.pallas import tpu as pltpu

N_DEV = 16
NCH = 4
NT = 8


def kernel(x, w_mat):
    m, _ = x.shape
    _, n = w_mat.shape
    h2 = m // 2
    h4 = m // 4
    h8 = m // 8
    ch = m // N_DEV
    cw = n // NCH

    def body(x_ref, w_ref, out_ref, sbuf, rbx, rby, rbz1, rbz2, rbza2, rbza1,
             rbya, rbxa, send_sems, recv_sems):
        p = lax.axis_index("i")
        q = lax.rem(p, 4)
        z = lax.div(p, 4)
        xbit = lax.rem(q, 2) ^ lax.div(q, 2)
        ybit = lax.div(q, 2)
        zb1 = lax.rem(z, 2)
        zb2 = lax.div(z, 2)
        px = p + 1 - 2 * lax.rem(q, 2)
        py = p + 3 - 2 * q
        pz1 = p + 4 - 8 * zb1
        pz2 = p + 8 - 16 * zb2

        ox = xbit * h2
        oxy = ox + ybit * h4
        oz1 = oxy + zb1 * h8
        oz2 = oz1 + zb2 * ch
        sx = (1 - xbit) * h2
        sy = ox + (1 - ybit) * h4
        sz1 = oxy + (1 - zb1) * h8
        sz2 = oz1 + (1 - zb2) * ch

        partners = (px, py, pz1, pz2)
        barrier = pltpu.get_barrier_semaphore()
        for nbr in partners:
            pl.semaphore_signal(
                barrier, inc=1, device_id=(nbr,),
                device_id_type=pl.DeviceIdType.MESH,
            )
        pl.semaphore_wait(barrier, len(partners))

        out_ref[:, :] = jnp.dot(
            x_ref[:, :], w_ref[:, :], preferred_element_type=jnp.float32
        )

        f32 = jnp.float32
        bf16 = jnp.bfloat16
        rdmas = {}

        plan = [
            (sx, h2, rbx, px, ox, True),
            (sy, h4, rby, py, oxy, True),
            (sz1, h8, rbz1, pz1, oz1, True),
            (sz2, ch, rbz2, pz2, oz2, True),
            (oz2, ch, rbza2, pz2, sz2, False),
            (oz1, h8, rbza1, pz1, sz1, False),
            (oxy, h4, rbya, py, sy, False),
            (ox, h2, rbxa, px, sx, False),
        ]

        def issue(c, k):
            rows, nrows, rbuf, peer, _, _ = plan[k]
            co = pl.ds(c * cw, cw)
            sbuf[c, k % 2, pl.ds(0, nrows), :] = out_ref[
                pl.ds(rows, nrows), co
            ].astype(bf16)
            d = pltpu.make_async_remote_copy(
                src_ref=sbuf.at[c, k % 2, pl.ds(0, nrows), :],
                dst_ref=rbuf.at[c],
                send_sem=send_sems.at[c, k], recv_sem=recv_sems.at[c, k],
                device_id=(peer,), device_id_type=pl.DeviceIdType.MESH,
            )
            d.start()
            rdmas[(c, k)] = d

        def apply(c, k):
            rdmas.pop((c, k)).wait()
            _, _, rbuf, _, rrows, reduce = plan[k]
            co = pl.ds(c * cw, cw)
            rows = pl.ds(rrows, plan[k][1])
            if reduce:
                out_ref[rows, co] = out_ref[rows, co] + rbuf[c].astype(f32)
            else:
                out_ref[rows, co] = rbuf[c].astype(f32)

        for t in range(NCH + NT):
            for c in range(NCH):
                k = t - c
                if 0 < k <= NT:
                    apply(c, k - 1)
                if 0 <= k < NT:
                    issue(c, k)

    return pl.pallas_call(
        body,
        out_shape=jax.ShapeDtypeStruct((m, n), jnp.float32),
        in_specs=[
            pl.BlockSpec(memory_space=pltpu.VMEM),
            pl.BlockSpec(memory_space=pltpu.VMEM),
        ],
        out_specs=pl.BlockSpec(memory_space=pltpu.VMEM),
        scratch_shapes=[
            pltpu.VMEM((NCH, 2, h2, cw), jnp.bfloat16),
            pltpu.VMEM((NCH, h2, cw), jnp.bfloat16),
            pltpu.VMEM((NCH, h4, cw), jnp.bfloat16),
            pltpu.VMEM((NCH, h8, cw), jnp.bfloat16),
            pltpu.VMEM((NCH, ch, cw), jnp.bfloat16),
            pltpu.VMEM((NCH, ch, cw), jnp.bfloat16),
            pltpu.VMEM((NCH, h8, cw), jnp.bfloat16),
            pltpu.VMEM((NCH, h4, cw), jnp.bfloat16),
            pltpu.VMEM((NCH, h2, cw), jnp.bfloat16),
            pltpu.SemaphoreType.DMA((NCH, NT)),
            pltpu.SemaphoreType.DMA((NCH, NT)),
        ],
        compiler_params=pltpu.CompilerParams(collective_id=0),
    )(x, w_mat)


# device time: 52041 ns/iter; 2.8396x vs baseline; 1.0078x over previous
import jax
import jax.numpy as jnp
from jax import lax
from jax.experimental import pallas as pl
from jax.experimental.pallas import tpu as pltpu

N_DEV = 16
NCH = 4
NT = 8


def kernel(x, w_mat):
    m, _ = x.shape
    _, n = w_mat.shape
    h2 = m // 2
    h4 = m // 4
    h8 = m // 8
    ch = m // N_DEV
    cw = n // NCH

    def body(x_ref, w_ref, out_ref, acc, rbx, rby, rbz1, rbz2,
             send_sems, recv_sems):
        p = lax.axis_index("i")
        q = lax.rem(p, 4)
        z = lax.div(p, 4)
        xbit = lax.rem(q, 2) ^ lax.div(q, 2)
        ybit = lax.div(q, 2)
        zb1 = lax.rem(z, 2)
        zb2 = lax.div(z, 2)
        px = p + 1 - 2 * lax.rem(q, 2)
        py = p + 3 - 2 * q
        pz1 = p + 4 - 8 * zb1
        pz2 = p + 8 - 16 * zb2

        ox = xbit * h2
        oxy = ox + ybit * h4
        oz1 = oxy + zb1 * h8
        oz2 = oz1 + zb2 * ch
        sx = (1 - xbit) * h2
        sy = ox + (1 - ybit) * h4
        sz1 = oxy + (1 - zb1) * h8
        sz2 = oz1 + (1 - zb2) * ch

        partners = (px, py, pz1, pz2)
        barrier = pltpu.get_barrier_semaphore()
        for nbr in partners:
            pl.semaphore_signal(
                barrier, inc=1, device_id=(nbr,),
                device_id_type=pl.DeviceIdType.MESH,
            )
        pl.semaphore_wait(barrier, len(partners))

        acc[:, :] = jnp.dot(
            x_ref[:, :], w_ref[:, :], preferred_element_type=jnp.float32
        ).astype(jnp.bfloat16)

        rdmas = {}

        plan = [
            (sx, h2, rbx, px, ox),
            (sy, h4, rby, py, oxy),
            (sz1, h8, rbz1, pz1, oz1),
            (sz2, ch, rbz2, pz2, oz2),
            (oz2, ch, None, pz2, None),
            (oz1, h8, None, pz1, None),
            (oxy, h4, None, py, None),
            (ox, h2, None, px, None),
        ]

        def issue(c, k):
            rows, nrows, rbuf, peer, _ = plan[k]
            src = acc.at[pl.ds(rows, nrows), pl.ds(c * cw, cw)]
            dst = src if rbuf is None else rbuf.at[c]
            d = pltpu.make_async_remote_copy(
                src_ref=src, dst_ref=dst,
                send_sem=send_sems.at[c, k], recv_sem=recv_sems.at[c, k],
                device_id=(peer,), device_id_type=pl.DeviceIdType.MESH,
            )
            d.start()
            rdmas[(c, k)] = d

        def apply(c, k):
            rdmas.pop((c, k)).wait()
            _, nrows, rbuf, _, rrows = plan[k]
            if rbuf is not None:
                rows = pl.ds(rrows, nrows)
                co = pl.ds(c * cw, cw)
                acc[rows, co] = acc[rows, co] + rbuf[c]

        for t in range(NCH + NT):
            for c in range(NCH):
                k = t - c
                if 0 < k <= NT:
                    apply(c, k - 1)
                if k == NT:
                    co = pl.ds(c * cw, cw)
                    out_ref[:, co] = acc[:, co].astype(jnp.float32)
                if 0 <= k < NT:
                    issue(c, k)

    return pl.pallas_call(
        body,
        out_shape=jax.ShapeDtypeStruct((m, n), jnp.float32),
        in_specs=[
            pl.BlockSpec(memory_space=pltpu.VMEM),
            pl.BlockSpec(memory_space=pltpu.VMEM),
        ],
        out_specs=pl.BlockSpec(memory_space=pltpu.VMEM),
        scratch_shapes=[
            pltpu.VMEM((m, n), jnp.bfloat16),
            pltpu.VMEM((NCH, h2, cw), jnp.bfloat16),
            pltpu.VMEM((NCH, h4, cw), jnp.bfloat16),
            pltpu.VMEM((NCH, h8, cw), jnp.bfloat16),
            pltpu.VMEM((NCH, ch, cw), jnp.bfloat16),
            pltpu.SemaphoreType.DMA((NCH, NT)),
            pltpu.SemaphoreType.DMA((NCH, NT)),
        ],
        compiler_params=pltpu.CompilerParams(collective_id=0),
    )(x, w_mat)


# device time: 44808 ns/iter; 3.2980x vs baseline; 1.1614x over previous
import os

import jax
import jax.numpy as jnp
from jax import lax
from jax.experimental import pallas as pl
from jax.experimental.pallas import tpu as pltpu

N_DEV = 16
NG = 2
NSC = 2
NU = NG * NSC
NT = 8
_NOCOMM = os.environ.get("KERNEL_NOCOMM") == "1"


def kernel(x, w_mat):
    m, _ = x.shape
    _, n = w_mat.shape
    cw = n // NU
    hs = [m // 2, m // 4, m // 8, m // 16]

    def body(x_ref, w_ref, out_ref, acc, rb0, rb1, rb2, rb3,
             send_sems, recv_sems):
        p = lax.axis_index("i")
        q = lax.rem(p, 4)
        z = lax.div(p, 4)
        dims = {
            "x": (lax.rem(q, 2) ^ lax.div(q, 2), p + 1 - 2 * lax.rem(q, 2)),
            "y": (lax.div(q, 2), p + 3 - 2 * q),
            "z1": (lax.rem(z, 2), p + 4 - 8 * lax.rem(z, 2)),
            "z2": (lax.div(z, 2), p + 8 - 16 * lax.div(z, 2)),
        }
        orders = (("x", "y", "z1", "z2"), ("y", "x", "z2", "z1"))
        rbufs = [rb0, rb1, rb2, rb3]

        partners = tuple(dims[d][1] for d in orders[0])
        barrier = pltpu.get_barrier_semaphore()
        for nbr in partners:
            pl.semaphore_signal(
                barrier, inc=1, device_id=(nbr,),
                device_id_type=pl.DeviceIdType.MESH,
            )
        pl.semaphore_wait(barrier, len(partners))

        acc[:, :] = jnp.dot(
            x_ref[:, :], w_ref[:, :], preferred_element_type=jnp.float32
        ).astype(jnp.bfloat16)

        plans = []
        for g in range(NG):
            plan = []
            o = 0
            offs = [0]
            for l, dname in enumerate(orders[g]):
                bit, peer = dims[dname]
                h = hs[l]
                plan.append((o + (1 - bit) * h, h, l, peer, o + bit * h))
                o = o + bit * h
                offs.append(o)
            for j in range(4):
                l = 3 - j
                _, peer = dims[orders[g][l]]
                plan.append((offs[l + 1], hs[l], None, peer, None))
            plans.append(plan)

        rdmas = {}

        def issue(u, k):
            rows, nrows, lvl, peer, _ = plans[u // NSC][k]
            src = acc.at[pl.ds(rows, nrows), pl.ds(u * cw, cw)]
            dst = src if lvl is None else rbufs[lvl].at[u]
            d = pltpu.make_async_remote_copy(
                src_ref=src, dst_ref=dst,
                send_sem=send_sems.at[u, k], recv_sem=recv_sems.at[u, k],
                device_id=(peer,), device_id_type=pl.DeviceIdType.MESH,
            )
            d.start()
            rdmas[(u, k)] = d

        def apply(u, k):
            rdmas.pop((u, k)).wait()
            _, nrows, lvl, _, aoff = plans[u // NSC][k]
            if lvl is not None:
                rows = pl.ds(aoff, nrows)
                co = pl.ds(u * cw, cw)
                acc[rows, co] = acc[rows, co] + rbufs[lvl][u]

        for t in range(NSC + NT):
            for u in range(NU):
                k = t - (u % NSC)
                if not _NOCOMM:
                    if 0 < k <= NT:
                        apply(u, k - 1)
                if k == NT:
                    co = pl.ds(u * cw, cw)
                    out_ref[:, co] = acc[:, co].astype(jnp.float32)
                if not _NOCOMM:
                    if 0 <= k < NT:
                        issue(u, k)

    return pl.pallas_call(
        body,
        out_shape=jax.ShapeDtypeStruct((m, n), jnp.float32),
        in_specs=[
            pl.BlockSpec(memory_space=pltpu.VMEM),
            pl.BlockSpec(memory_space=pltpu.VMEM),
        ],
        out_specs=pl.BlockSpec(memory_space=pltpu.VMEM),
        scratch_shapes=[
            pltpu.VMEM((m, n), jnp.bfloat16),
            pltpu.VMEM((NU, hs[0], cw), jnp.bfloat16),
            pltpu.VMEM((NU, hs[1], cw), jnp.bfloat16),
            pltpu.VMEM((NU, hs[2], cw), jnp.bfloat16),
            pltpu.VMEM((NU, hs[3], cw), jnp.bfloat16),
            pltpu.SemaphoreType.DMA((NU, NT)),
            pltpu.SemaphoreType.DMA((NU, NT)),
        ],
        compiler_params=pltpu.CompilerParams(collective_id=0),
    )(x, w_mat)


# device time: 43106 ns/iter; 3.4282x vs baseline; 1.0395x over previous
import os

import jax
import jax.numpy as jnp
from jax import lax
from jax.experimental import pallas as pl
from jax.experimental.pallas import tpu as pltpu

N_DEV = 16
NG = 2
NSC = int(os.environ.get("KERNEL_NSC", "4"))
NU = NG * NSC
NT = 8
_NOCOMM = os.environ.get("KERNEL_NOCOMM") == "1"


def kernel(x, w_mat):
    m, _ = x.shape
    _, n = w_mat.shape
    cw = n // NU
    hs = [m // 2, m // 4, m // 8, m // 16]

    def body(x_ref, w_ref, out_ref, acc, rb0, rb1, rb2, rb3,
             send_sems, recv_sems):
        p = lax.axis_index("i")
        q = lax.rem(p, 4)
        z = lax.div(p, 4)
        dims = {
            "x": (lax.rem(q, 2) ^ lax.div(q, 2), p + 1 - 2 * lax.rem(q, 2)),
            "y": (lax.div(q, 2), p + 3 - 2 * q),
            "z1": (lax.rem(z, 2), p + 4 - 8 * lax.rem(z, 2)),
            "z2": (lax.div(z, 2), p + 8 - 16 * lax.div(z, 2)),
        }
        orders = (("x", "y", "z1", "z2"), ("y", "x", "z2", "z1"))
        rbufs = [rb0, rb1, rb2, rb3]

        partners = tuple(dims[d][1] for d in orders[0])
        barrier = pltpu.get_barrier_semaphore()
        for nbr in partners:
            pl.semaphore_signal(
                barrier, inc=1, device_id=(nbr,),
                device_id_type=pl.DeviceIdType.MESH,
            )
        pl.semaphore_wait(barrier, len(partners))

        acc[:, :] = jnp.dot(
            x_ref[:, :], w_ref[:, :], preferred_element_type=jnp.float32
        ).astype(jnp.bfloat16)

        plans = []
        for g in range(NG):
            plan = []
            o = 0
            offs = [0]
            for l, dname in enumerate(orders[g]):
                bit, peer = dims[dname]
                h = hs[l]
                plan.append((o + (1 - bit) * h, h, l, peer, o + bit * h))
                o = o + bit * h
                offs.append(o)
            for j in range(4):
                l = 3 - j
                _, peer = dims[orders[g][l]]
                plan.append((offs[l + 1], hs[l], None, peer, None))
            plans.append(plan)

        rdmas = {}

        def issue(u, k):
            rows, nrows, lvl, peer, _ = plans[u // NSC][k]
            src = acc.at[pl.ds(rows, nrows), pl.ds(u * cw, cw)]
            dst = src if lvl is None else rbufs[lvl].at[u]
            d = pltpu.make_async_remote_copy(
                src_ref=src, dst_ref=dst,
                send_sem=send_sems.at[u, k], recv_sem=recv_sems.at[u, k],
                device_id=(peer,), device_id_type=pl.DeviceIdType.MESH,
            )
            d.start()
            rdmas[(u, k)] = d

        def apply(u, k):
            rdmas.pop((u, k)).wait()
            _, nrows, lvl, _, aoff = plans[u // NSC][k]
            if lvl is not None:
                rows = pl.ds(aoff, nrows)
                co = pl.ds(u * cw, cw)
                acc[rows, co] = acc[rows, co] + rbufs[lvl][u]

        for t in range(NSC + NT):
            for u in range(NU):
                k = t - (u % NSC)
                if not _NOCOMM:
                    if 0 < k <= NT:
                        apply(u, k - 1)
                if k == NT:
                    co = pl.ds(u * cw, cw)
                    out_ref[:, co] = acc[:, co].astype(jnp.float32)
                if not _NOCOMM:
                    if 0 <= k < NT:
                        issue(u, k)

    return pl.pallas_call(
        body,
        out_shape=jax.ShapeDtypeStruct((m, n), jnp.float32),
        in_specs=[
            pl.BlockSpec(memory_space=pltpu.VMEM),
            pl.BlockSpec(memory_space=pltpu.VMEM),
        ],
        out_specs=pl.BlockSpec(memory_space=pltpu.VMEM),
        scratch_shapes=[
            pltpu.VMEM((m, n), jnp.bfloat16),
            pltpu.VMEM((NU, hs[0], cw), jnp.bfloat16),
            pltpu.VMEM((NU, hs[1], cw), jnp.bfloat16),
            pltpu.VMEM((NU, hs[2], cw), jnp.bfloat16),
            pltpu.VMEM((NU, hs[3], cw), jnp.bfloat16),
            pltpu.SemaphoreType.DMA((NU, NT)),
            pltpu.SemaphoreType.DMA((NU, NT)),
        ],
        compiler_params=pltpu.CompilerParams(collective_id=0),
    )(x, w_mat)


# device time: 40079 ns/iter; 3.6871x vs baseline; 1.0755x over previous
import os

import jax
import jax.numpy as jnp
from jax import lax
from jax.experimental import pallas as pl
from jax.experimental.pallas import tpu as pltpu

N_DEV = 16
NG = 2
NSC = int(os.environ.get("KERNEL_NSC", "4"))
NU = NG * NSC
NT = 6
NSLOT = 10
_NOCOMM = os.environ.get("KERNEL_NOCOMM") == "1"


def kernel(x, w_mat):
    m, _ = x.shape
    _, n = w_mat.shape
    cw = n // NU
    h2, h4 = m // 2, m // 4
    ch = m // N_DEV

    def body(x_ref, w_ref, out_ref, acc, rb0, rb1, rbz, send_sems, recv_sems):
        p = lax.axis_index("i")
        q = lax.rem(p, 4)
        z = lax.div(p, 4)
        dims = {
            "x": (lax.rem(q, 2) ^ lax.div(q, 2), p + 1 - 2 * lax.rem(q, 2)),
            "y": (lax.div(q, 2), p + 3 - 2 * q),
        }
        zpeers = [lax.rem(p + 4 * d, N_DEV) for d in (1, 2, 3)]
        orders = (("x", "y"), ("y", "x"))

        partners = (dims["x"][1], dims["y"][1], *zpeers)
        barrier = pltpu.get_barrier_semaphore()
        for nbr in partners:
            pl.semaphore_signal(
                barrier, inc=1, device_id=(nbr,),
                device_id_type=pl.DeviceIdType.MESH,
            )
        pl.semaphore_wait(barrier, len(partners))

        acc[:, :] = jnp.dot(
            x_ref[:, :], w_ref[:, :], preferred_element_type=jnp.float32
        ).astype(jnp.bfloat16)

        geo = []
        for g in range(NG):
            b0, p0 = dims[orders[g][0]]
            b1, p1 = dims[orders[g][1]]
            o1 = b0 * h2
            geo.append((
                (1 - b0) * h2, o1, o1 + (1 - b1) * h4, o1 + b1 * h4, p0, p1,
            ))

        rdmas = {}

        def rc(u, slot, src, dst, peer):
            d = pltpu.make_async_remote_copy(
                src_ref=src, dst_ref=dst,
                send_sem=send_sems.at[u, slot], recv_sem=recv_sems.at[u, slot],
                device_id=(peer,), device_id_type=pl.DeviceIdType.MESH,
            )
            d.start()
            return d

        def issue(u, k):
            s0, o1, s1, oxy, p0, p1 = geo[u // NSC]
            co = pl.ds(u * cw, cw)
            my_chunk = pl.ds(oxy + z * ch, ch)
            if k == 0:
                src = acc.at[pl.ds(s0, h2), co]
                rdmas[(u, k)] = [rc(u, 0, src, rb0.at[u], p0)]
            elif k == 1:
                src = acc.at[pl.ds(s1, h4), co]
                rdmas[(u, k)] = [rc(u, 1, src, rb1.at[u], p1)]
            elif k == 2:
                ds_ = []
                for d in (1, 2, 3):
                    cz = lax.rem(z + d, 4)
                    src = acc.at[pl.ds(oxy + cz * ch, ch), co]
                    ds_.append(rc(u, 1 + d, src, rbz.at[u, d - 1], zpeers[d - 1]))
                rdmas[(u, k)] = ds_
            elif k == 3:
                src = acc.at[my_chunk, co]
                rdmas[(u, k)] = [
                    rc(u, 4 + d, src, src, zpeers[d - 1]) for d in (1, 2, 3)
                ]
            elif k == 4:
                src = acc.at[pl.ds(oxy, h4), co]
                rdmas[(u, k)] = [rc(u, 8, src, src, p1)]
            else:
                src = acc.at[pl.ds(o1, h2), co]
                rdmas[(u, k)] = [rc(u, 9, src, src, p0)]

        def apply(u, k):
            for d in rdmas.pop((u, k)):
                d.wait()
            s0, o1, s1, oxy, _, _ = geo[u // NSC]
            co = pl.ds(u * cw, cw)
            if k == 0:
                rows = pl.ds(o1, h2)
                acc[rows, co] = acc[rows, co] + rb0[u]
            elif k == 1:
                rows = pl.ds(oxy, h4)
                acc[rows, co] = acc[rows, co] + rb1[u]
            elif k == 2:
                rows = pl.ds(oxy + z * ch, ch)
                acc[rows, co] = (
                    acc[rows, co] + rbz[u, 0] + rbz[u, 1] + rbz[u, 2]
                )

        for t in range(NSC + NT):
            for u in range(NU):
                k = t - (u % NSC)
                if not _NOCOMM:
                    if 0 < k <= NT:
                        apply(u, k - 1)
                if k == NT:
                    co = pl.ds(u * cw, cw)
                    out_ref[:, co] = acc[:, co].astype(jnp.float32)
                if not _NOCOMM:
                    if 0 <= k < NT:
                        issue(u, k)

    return pl.pallas_call(
        body,
        out_shape=jax.ShapeDtypeStruct((m, n), jnp.float32),
        in_specs=[
            pl.BlockSpec(memory_space=pltpu.VMEM),
            pl.BlockSpec(memory_space=pltpu.VMEM),
        ],
        out_specs=pl.BlockSpec(memory_space=pltpu.VMEM),
        scratch_shapes=[
            pltpu.VMEM((m, n), jnp.bfloat16),
            pltpu.VMEM((NU, h2, cw), jnp.bfloat16),
            pltpu.VMEM((NU, h4, cw), jnp.bfloat16),
            pltpu.VMEM((NU, 3, ch, cw), jnp.bfloat16),
            pltpu.SemaphoreType.DMA((NU, NSLOT)),
            pltpu.SemaphoreType.DMA((NU, NSLOT)),
        ],
        compiler_params=pltpu.CompilerParams(collective_id=0),
    )(x, w_mat)
